# Initial kernel scaffold; baseline (speedup 1.0000x reference)
import jax
import jax.numpy as jnp
from jax.experimental import pallas as pl


def kernel(x, edge_index, edge_w0, edge_w1, W):
    raise NotImplementedError("write your pallas kernel here")


# speedup vs baseline: 5.5527x; 5.5527x over previous
import functools

import jax
import jax.numpy as jnp
from jax import lax
from jax.experimental import pallas as pl
from jax.experimental.pallas import tpu as pltpu
from jax.experimental.pallas import tpu_sc as plsc

N = 10000
E = 160000
D = 256
H = D // 2

NSUB = 16
EPS = E // NSUB
K = 80
NCH = EPS // K
ACC_N = 10240
RPS = ACC_N // NSUB
CPB = 25
NB = NCH // CPB


def _tc_features(x_blk, w_ref, o_ref):
    p = jnp.dot(x_blk[...], w_ref[...], preferred_element_type=jnp.float32)
    m = p[:, :H]
    v = p[:, H:]
    mv = jnp.where(m > 0, m, jnp.exp(jnp.minimum(m, 0.0)) - 1.0)
    vv = jnp.maximum(v, 0.0)
    nw = jnp.exp(-vv)
    o_ref[0] = mv * nw
    o_ref[1] = vv * (nw * nw)


def _features(x, W):
    blk = 1000
    grid = N // blk
    return pl.pallas_call(
        _tc_features,
        grid=(grid,),
        in_specs=[
            pl.BlockSpec((blk, D), lambda i: (i, 0)),
            pl.BlockSpec((D, D), lambda i: (0, 0)),
        ],
        out_specs=pl.BlockSpec((2, blk, H), lambda i: (0, i, 0)),
        out_shape=jax.ShapeDtypeStruct((2, N, H), jnp.float32),
    )(x, W)


def _sc_aggregate(f2, srcs, dsts, ws):
    mesh = plsc.VectorSubcoreMesh(core_axis_name="c", subcore_axis_name="s")

    @functools.partial(
        pl.kernel,
        mesh=mesh,
        out_type=jax.ShapeDtypeStruct((2 * ACC_N, H), jnp.float32),
        scratch_types=[
            pltpu.VMEM((CPB, K), jnp.int32),
            pltpu.VMEM((CPB, K), jnp.int32),
            pltpu.VMEM((CPB, K), jnp.float32),
            pltpu.VMEM((K, H), jnp.float32),
            pltpu.VMEM_SHARED((ACC_N, H), jnp.float32),
            pltpu.SemaphoreType.DMA,
        ],
    )
    def agg(f_hbm, src_hbm, dst_hbm, w_hbm, out_hbm,
            src_v, dst_v, w_v, rows_v, acc_sh, sem):
        c = lax.axis_index("c")
        s = lax.axis_index("s")

        z16 = jnp.zeros((16,), jnp.float32)

        def zrow(i, _):
            r = i // (H // 16)
            j = i - r * (H // 16)
            rows_v[r, pl.ds(j * 16, 16)] = z16
            return 0

        lax.fori_loop(0, K * (H // 16), zrow, 0)
        for t in range(RPS // K):
            pltpu.sync_copy(rows_v, acc_sh.at[pl.ds(s * RPS + t * K, K)])
        plsc.subcore_barrier()

        def chunk(i, _):
            pltpu.async_copy(f_hbm.at[src_v.at[i]], rows_v, sem).wait()

            for g in range(K // 16):
                wv16 = w_v[i, pl.ds(g * 16, 16)]
                for t in range(16):
                    e = g * 16 + t
                    wb = jnp.full((16,), wv16[t], jnp.float32)
                    for j in range(H // 16):
                        sl = pl.ds(j * 16, 16)
                        rows_v[e, sl] = rows_v[e, sl] * wb

            pltpu.sync_copy(rows_v, acc_sh.at[dst_v.at[i]], add=True)
            return 0

        for b in range(NB):
            pltpu.sync_copy(src_hbm.at[c, s, b], src_v)
            pltpu.sync_copy(dst_hbm.at[s, b], dst_v)
            pltpu.sync_copy(w_hbm.at[c, s, b], w_v)
            lax.fori_loop(0, CPB, chunk, 0)
        plsc.subcore_barrier()

        pltpu.sync_copy(
            acc_sh.at[pl.ds(s * RPS, RPS)],
            out_hbm.at[pl.ds(c * ACC_N + s * RPS, RPS)],
        )

    return agg(f2, srcs, dsts, ws)


def kernel(x, edge_index, edge_w0, edge_w1, W):
    f = _features(x, W).reshape(2 * N, H)

    src = edge_index[0]
    dst = edge_index[1]
    srcs = jnp.stack([src, src + N]).reshape(2, NSUB, NB, CPB, K)
    dsts = dst.reshape(NSUB, NB, CPB, K)
    ws = jnp.stack([edge_w0, edge_w1]).reshape(2, NSUB, NB, CPB, K)

    out = _sc_aggregate(f, srcs, dsts, ws)
    return jnp.concatenate([out[:N], out[ACC_N:ACC_N + N]], axis=1)


# speedup vs baseline: 8.9777x; 1.6168x over previous
import functools

import jax
import jax.numpy as jnp
from jax import lax
from jax.experimental import pallas as pl
from jax.experimental.pallas import tpu as pltpu
from jax.experimental.pallas import tpu_sc as plsc

N = 10000
E = 160000
D = 256
H = D // 2

NSUB = 16
EPS = E // NSUB
K = 80
NCH = EPS // K
ACC_N = 10240
RPS = ACC_N // NSUB
CPB = 25
NB = NCH // CPB
TPB = (CPB - 4) // 3


def _tc_features(x_blk, w_ref, o_ref):
    p = jnp.dot(x_blk[...], w_ref[...], preferred_element_type=jnp.float32)
    m = p[:, :H]
    v = p[:, H:]
    mv = jnp.where(m > 0, m, jnp.exp(jnp.minimum(m, 0.0)) - 1.0)
    vv = jnp.maximum(v, 0.0)
    nw = jnp.exp(-vv)
    o_ref[0] = mv * nw
    o_ref[1] = vv * (nw * nw)


def _features(x, W):
    blk = 1000
    grid = N // blk
    return pl.pallas_call(
        _tc_features,
        grid=(grid,),
        in_specs=[
            pl.BlockSpec((blk, D), lambda i: (i, 0)),
            pl.BlockSpec((D, D), lambda i: (0, 0)),
        ],
        out_specs=pl.BlockSpec((2, blk, H), lambda i: (0, i, 0)),
        out_shape=jax.ShapeDtypeStruct((2, N, H), jnp.float32),
    )(x, W)


def _sc_aggregate(f2, srcs, dsts, ws):
    mesh = plsc.VectorSubcoreMesh(core_axis_name="c", subcore_axis_name="s")

    @functools.partial(
        pl.kernel,
        mesh=mesh,
        out_type=jax.ShapeDtypeStruct((2 * ACC_N, H), jnp.float32),
        scratch_types=[
            pltpu.VMEM((CPB, K), jnp.int32),
            pltpu.VMEM((CPB, K), jnp.int32),
            pltpu.VMEM((CPB, K), jnp.float32),
            pltpu.VMEM((K, H), jnp.float32),
            pltpu.VMEM((K, H), jnp.float32),
            pltpu.VMEM((K, H), jnp.float32),
            pltpu.VMEM_SHARED((ACC_N, H), jnp.float32),
            pltpu.SemaphoreType.DMA,
            pltpu.SemaphoreType.DMA,
            pltpu.SemaphoreType.DMA,
            pltpu.SemaphoreType.DMA,
            pltpu.SemaphoreType.DMA,
            pltpu.SemaphoreType.DMA,
        ],
    )
    def agg(f_hbm, src_hbm, dst_hbm, w_hbm, out_hbm,
            src_v, dst_v, w_v, bufa, bufb, bufc, acc_sh,
            ga, gb, gc, sa, sb, sc_):
        c = lax.axis_index("c")
        s = lax.axis_index("s")

        z16 = jnp.zeros((16,), jnp.float32)

        def zrow(i, _):
            r = i // (H // 16)
            j = i - r * (H // 16)
            bufa[r, pl.ds(j * 16, 16)] = z16
            return 0

        lax.fori_loop(0, K * (H // 16), zrow, 0)
        for t in range(RPS // K):
            pltpu.sync_copy(bufa, acc_sh.at[pl.ds(s * RPS + t * K, K)])
        plsc.subcore_barrier()

        def gather(i, buf, gsem):
            pltpu.async_copy(f_hbm.at[src_v.at[i]], buf, gsem)

        def gwait(buf, gsem):
            pltpu.make_async_copy(f_hbm.at[src_v.at[0]], buf, gsem).wait()

        def scatter(i, buf, ssem):
            pltpu.async_copy(buf, acc_sh.at[dst_v.at[i]], ssem, add=True)

        def swait(buf, ssem):
            pltpu.make_async_copy(buf, acc_sh.at[dst_v.at[0]], ssem).wait()

        def scale(i, buf):
            def grp(g, _):
                wv16 = w_v[i, pl.ds(g * 16, 16)]
                for t in range(16):
                    e = g * 16 + t
                    wb = jnp.full((16,), wv16[t], jnp.float32)
                    for j in range(H // 16):
                        sl = pl.ds(j * 16, 16)
                        buf[e, sl] = buf[e, sl] * wb
                return 0

            lax.fori_loop(0, K // 16, grp, 0)

        def block(b, _):
            pltpu.sync_copy(src_hbm.at[c, s, b], src_v)
            pltpu.sync_copy(dst_hbm.at[s, b], dst_v)
            pltpu.sync_copy(w_hbm.at[c, s, b], w_v)

            gather(0, bufa, ga)
            gather(1, bufb, gb)
            gather(2, bufc, gc)

            gwait(bufa, ga)
            scale(0, bufa)
            scatter(0, bufa, sa)
            gwait(bufb, gb)
            scale(1, bufb)
            scatter(1, bufb, sb)
            swait(bufa, sa)
            gather(3, bufa, ga)
            gwait(bufc, gc)
            scale(2, bufc)
            scatter(2, bufc, sc_)
            swait(bufb, sb)
            gather(4, bufb, gb)

            def trip(p, _):
                i0 = 3 * p
                swait(bufc, sc_)
                gather(i0 + 2, bufc, gc)
                gwait(bufa, ga)
                scale(i0, bufa)
                scatter(i0, bufa, sa)
                gwait(bufb, gb)
                scale(i0 + 1, bufb)
                scatter(i0 + 1, bufb, sb)
                swait(bufa, sa)
                gather(i0 + 3, bufa, ga)
                gwait(bufc, gc)
                scale(i0 + 2, bufc)
                scatter(i0 + 2, bufc, sc_)
                swait(bufb, sb)
                gather(i0 + 4, bufb, gb)
                return 0

            lax.fori_loop(1, TPB, trip, 0)

            i0 = 3 * TPB
            swait(bufc, sc_)
            gather(i0 + 2, bufc, gc)
            gwait(bufa, ga)
            scale(i0, bufa)
            scatter(i0, bufa, sa)
            gwait(bufb, gb)
            scale(i0 + 1, bufb)
            scatter(i0 + 1, bufb, sb)
            swait(bufa, sa)
            gather(i0 + 3, bufa, ga)
            gwait(bufc, gc)
            scale(i0 + 2, bufc)
            scatter(i0 + 2, bufc, sc_)

            gwait(bufa, ga)
            scale(CPB - 1, bufa)
            scatter(CPB - 1, bufa, sa)

            swait(bufa, sa)
            swait(bufb, sb)
            swait(bufc, sc_)
            return 0

        lax.fori_loop(0, NB, block, 0)
        plsc.subcore_barrier()

        pltpu.sync_copy(
            acc_sh.at[pl.ds(s * RPS, RPS)],
            out_hbm.at[pl.ds(c * ACC_N + s * RPS, RPS)],
        )

    return agg(f2, srcs, dsts, ws)


def kernel(x, edge_index, edge_w0, edge_w1, W):
    f = _features(x, W).reshape(2 * N, H)

    src = edge_index[0]
    dst = edge_index[1]
    srcs = jnp.stack([src, src + N]).reshape(2, NSUB, NB, CPB, K)
    dsts = dst.reshape(NSUB, NB, CPB, K)
    ws = jnp.stack([edge_w0, edge_w1]).reshape(2, NSUB, NB, CPB, K)

    out = _sc_aggregate(f, srcs, dsts, ws)
    return jnp.concatenate([out[:N], out[ACC_N:ACC_N + N]], axis=1)
